# Initial kernel scaffold; baseline (speedup 1.0000x reference)
import jax
import jax.numpy as jnp
from jax.experimental import pallas as pl


def kernel(t, pe):
    raise NotImplementedError("write your pallas kernel here")


# speedup vs baseline: 3.2200x; 3.2200x over previous
import functools

import jax
import jax.numpy as jnp
from jax import lax
from jax.experimental import pallas as pl
from jax.experimental.pallas import tpu as pltpu
from jax.experimental.pallas import tpu_sc as plsc

D = 128
B = 4096 * 50
NC, NS = 2, 16
NW = NC * NS
B_PER_W = B // NW
CHUNK = 128
NCHUNK = B_PER_W // CHUNK

_mesh = plsc.VectorSubcoreMesh(core_axis_name="c", subcore_axis_name="s")


@functools.partial(
    pl.kernel,
    mesh=_mesh,
    out_type=jax.ShapeDtypeStruct((B, D), jnp.float32),
    scratch_types=[
        pltpu.VMEM((B_PER_W,), jnp.int32),
        pltpu.VMEM((CHUNK, D), jnp.float32),
        pltpu.SemaphoreType.DMA,
    ],
)
def _gather_kernel(pe_hbm, idx_hbm, out_hbm, idx_v, buf, gsem):
    wid = lax.axis_index("s") * NC + lax.axis_index("c")
    base = wid * B_PER_W
    pltpu.sync_copy(idx_hbm.at[pl.ds(base, B_PER_W)], idx_v)

    def body(j, carry):
        pltpu.async_copy(
            pe_hbm.at[idx_v.at[pl.ds(j * CHUNK, CHUNK)]], buf, gsem
        ).wait()
        pltpu.sync_copy(buf, out_hbm.at[pl.ds(base + j * CHUNK, CHUNK)])
        return carry

    lax.fori_loop(0, NCHUNK, body, 0)


def kernel(t, pe):
    out = _gather_kernel(pe, t.reshape(-1))
    return out.reshape(t.shape + (D,))


# speedup vs baseline: 3.5901x; 1.1149x over previous
import functools

import jax
import jax.numpy as jnp
from jax import lax
from jax.experimental import pallas as pl
from jax.experimental.pallas import tpu as pltpu
from jax.experimental.pallas import tpu_sc as plsc

D = 128
B = 4096 * 50
NC, NS = 2, 16
NW = NC * NS
B_PER_W = B // NW
CHUNK = 128
NCHUNK = B_PER_W // CHUNK
NBUF = 5
NGRP = NCHUNK // NBUF

_mesh = plsc.VectorSubcoreMesh(core_axis_name="c", subcore_axis_name="s")


@functools.partial(
    pl.kernel,
    mesh=_mesh,
    out_type=jax.ShapeDtypeStruct((B, D), jnp.float32),
    scratch_types=[
        pltpu.VMEM((B_PER_W,), jnp.int32),
    ]
    + [pltpu.VMEM((CHUNK, D), jnp.float32) for _ in range(NBUF)]
    + [pltpu.SemaphoreType.DMA for _ in range(2 * NBUF)],
)
def _gather_kernel(pe_hbm, idx_hbm, out_hbm, idx_v, *rest):
    bufs = rest[:NBUF]
    gsems = rest[NBUF:2 * NBUF]
    ssems = rest[2 * NBUF:]

    wid = lax.axis_index("s") * NC + lax.axis_index("c")
    base = wid * B_PER_W
    pltpu.sync_copy(idx_hbm.at[pl.ds(base, B_PER_W)], idx_v)

    def gather(j, b):
        return pltpu.make_async_copy(
            pe_hbm.at[idx_v.at[pl.ds(j * CHUNK, CHUNK)]], bufs[b], gsems[b]
        )

    def store(j, b):
        return pltpu.make_async_copy(
            bufs[b], out_hbm.at[pl.ds(base + j * CHUNK, CHUNK)], ssems[b]
        )

    for b in range(NBUF):
        gather(b, b).start()

    def grp(g, carry):
        j0 = g * NBUF
        for b in range(NBUF):
            gather(j0 + b, b).wait()
            store(j0 + b, b).start()
        for b in range(NBUF):
            store(j0 + b, b).wait()
            gather(j0 + NBUF + b, b).start()
        return carry

    lax.fori_loop(0, NGRP - 1, grp, 0)

    j0 = (NGRP - 1) * NBUF
    for b in range(NBUF):
        gather(j0 + b, b).wait()
        store(j0 + b, b).start()
    for b in range(NBUF):
        store(j0 + b, b).wait()


def kernel(t, pe):
    out = _gather_kernel(pe, t.reshape(-1))
    return out.reshape(t.shape + (D,))


# speedup vs baseline: 3.7018x; 1.0311x over previous
import functools

import jax
import jax.numpy as jnp
from jax import lax
from jax.experimental import pallas as pl
from jax.experimental.pallas import tpu as pltpu
from jax.experimental.pallas import tpu_sc as plsc

D = 128
B = 4096 * 50
NC, NS = 2, 16
NW = NC * NS
B_PER_W = B // NW
CHUNK = 128
NCHUNK = B_PER_W // CHUNK
NBUF = 2
NGRP = NCHUNK // NBUF

_mesh = plsc.VectorSubcoreMesh(core_axis_name="c", subcore_axis_name="s")


@functools.partial(
    pl.kernel,
    mesh=_mesh,
    out_type=jax.ShapeDtypeStruct((B, D), jnp.float32),
    scratch_types=[
        pltpu.VMEM((B_PER_W,), jnp.int32),
        pltpu.VMEM_SHARED((10000, D), jnp.float32),
    ]
    + [pltpu.VMEM((CHUNK, D), jnp.float32) for _ in range(NBUF)]
    + [pltpu.SemaphoreType.DMA for _ in range(2 * NBUF)],
)
def _gather_kernel(pe_hbm, idx_hbm, out_hbm, idx_v, pe_sp, *rest):
    bufs = rest[:NBUF]
    gsems = rest[NBUF:2 * NBUF]
    ssems = rest[2 * NBUF:]

    sid = lax.axis_index("s")
    wid = sid * NC + lax.axis_index("c")
    base = wid * B_PER_W
    pltpu.sync_copy(idx_hbm.at[pl.ds(base, B_PER_W)], idx_v)

    rows = 624
    pltpu.sync_copy(
        pe_hbm.at[pl.ds(sid * rows, rows)], pe_sp.at[pl.ds(sid * rows, rows)]
    )

    @pl.when(sid == 0)
    def _():
        pltpu.sync_copy(
            pe_hbm.at[pl.ds(16 * rows, 10000 - 16 * rows)],
            pe_sp.at[pl.ds(16 * rows, 10000 - 16 * rows)],
        )

    plsc.subcore_barrier()

    def gather(j, b):
        return pltpu.make_async_copy(
            pe_sp.at[idx_v.at[pl.ds(j * CHUNK, CHUNK)]], bufs[b], gsems[b]
        )

    def store(j, b):
        return pltpu.make_async_copy(
            bufs[b], out_hbm.at[pl.ds(base + j * CHUNK, CHUNK)], ssems[b]
        )

    for b in range(NBUF):
        gather(b, b).start()

    def grp(g, carry):
        j0 = g * NBUF
        for b in range(NBUF):
            gather(j0 + b, b).wait()
            store(j0 + b, b).start()
        for b in range(NBUF):
            store(j0 + b, b).wait()
            gather(j0 + NBUF + b, b).start()
        return carry

    lax.fori_loop(0, NGRP - 1, grp, 0)

    j0 = (NGRP - 1) * NBUF
    for b in range(NBUF):
        gather(j0 + b, b).wait()
        store(j0 + b, b).start()
    for b in range(NBUF):
        store(j0 + b, b).wait()


def kernel(t, pe):
    out = _gather_kernel(pe, t.reshape(-1))
    return out.reshape(t.shape + (D,))


# speedup vs baseline: 7.7503x; 2.0937x over previous
import functools

import jax
import jax.numpy as jnp
from jax import lax
from jax.experimental import pallas as pl
from jax.experimental.pallas import tpu as pltpu
from jax.experimental.pallas import tpu_sc as plsc

D = 128
R = 4096
W = 50
V = 10000
NC, NS = 2, 16
NW = NC * NS
R_PER_W = R // NW
NBUF = 4
NGRP = R_PER_W // NBUF

_mesh = plsc.VectorSubcoreMesh(core_axis_name="c", subcore_axis_name="s")


@functools.partial(
    pl.kernel,
    mesh=_mesh,
    out_type=jax.ShapeDtypeStruct((R, W, D), jnp.float32),
    scratch_types=[
        pltpu.VMEM((R_PER_W, W), jnp.int32),
        pltpu.VMEM_SHARED((V, D), jnp.float32),
    ]
    + [pltpu.VMEM((W, D), jnp.float32) for _ in range(NBUF)]
    + [pltpu.SemaphoreType.DMA for _ in range(2 * NBUF)],
)
def _gather_kernel(pe_hbm, idx_hbm, out_hbm, idx_v, pe_sp, *rest):
    bufs = rest[:NBUF]
    gsems = rest[NBUF:2 * NBUF]
    ssems = rest[2 * NBUF:]

    sid = lax.axis_index("s")
    wid = sid * NC + lax.axis_index("c")
    base = wid * R_PER_W
    pltpu.sync_copy(idx_hbm.at[pl.ds(base, R_PER_W)], idx_v)

    rows = 624
    pltpu.sync_copy(
        pe_hbm.at[pl.ds(sid * rows, rows)], pe_sp.at[pl.ds(sid * rows, rows)]
    )

    @pl.when(sid == 0)
    def _():
        pltpu.sync_copy(
            pe_hbm.at[pl.ds(16 * rows, V - 16 * rows)],
            pe_sp.at[pl.ds(16 * rows, V - 16 * rows)],
        )

    plsc.subcore_barrier()

    def gather(r, b):
        return pltpu.make_async_copy(
            pe_sp.at[idx_v.at[r]], bufs[b], gsems[b]
        )

    def store(r, b):
        return pltpu.make_async_copy(bufs[b], out_hbm.at[base + r], ssems[b])

    for b in range(NBUF):
        gather(b, b).start()

    def grp(g, carry):
        r0 = g * NBUF
        for b in range(NBUF):
            gather(r0 + b, b).wait()
            store(r0 + b, b).start()
        for b in range(NBUF):
            store(r0 + b, b).wait()
            gather(r0 + NBUF + b, b).start()
        return carry

    lax.fori_loop(0, NGRP - 1, grp, 0)

    r0 = (NGRP - 1) * NBUF
    for b in range(NBUF):
        gather(r0 + b, b).wait()
        store(r0 + b, b).start()
    for b in range(NBUF):
        store(r0 + b, b).wait()


def kernel(t, pe):
    return _gather_kernel(pe, t)
